# baseline (speedup 1.0000x reference)
import functools

import jax
import jax.numpy as jnp
from jax import lax
from jax.experimental import pallas as pl
from jax.experimental.pallas import tpu as pltpu
from jax.experimental.pallas import tpu_sc as plsc

EMBED = 64
HIDDEN = 256
OUT = 2
N_TOKENS = 16384
VOCAB = 1000000

NC = 2
NS = 16
NW = NC * NS
PER_TILE = N_TOKENS // NW
CHUNK = 128
NCH = PER_TILE // CHUNK
LANES = 16
BINS = 1 << 20
SLICE = BINS // NS
BK = 23808
NBLK = 42
TAIL = NBLK * BK


def _sc_hist(idx, zeros):
    mesh = plsc.VectorSubcoreMesh(core_axis_name="c", subcore_axis_name="s")

    @functools.partial(
        pl.kernel,
        mesh=mesh,
        out_type=jax.ShapeDtypeStruct((NC, BINS), jnp.float32),
        scratch_types=[
            pltpu.VMEM((NCH, CHUNK), jnp.int32),
            pltpu.VMEM((NCH, CHUNK), jnp.float32),
            pltpu.VMEM_SHARED((BINS,), jnp.float32),
            pltpu.SemaphoreType.DMA,
        ],
    )
    def k(idx_hbm, zeros_hbm, out_hbm, idx_v, ones_v, shared, sem):
        cid = lax.axis_index("c")
        sid = lax.axis_index("s")
        wid = sid * NC + cid

        pltpu.sync_copy(idx_hbm.at[wid], idx_v)
        for c in range(NCH):
            for g in range(CHUNK // LANES):
                ones_v[c, pl.ds(g * LANES, LANES)] = jnp.ones((LANES,), jnp.float32)

        pltpu.sync_copy(
            zeros_hbm.at[cid, pl.ds(sid * SLICE, SLICE)],
            shared.at[pl.ds(sid * SLICE, SLICE)],
        )
        plsc.subcore_barrier()

        for c in range(NCH):
            pltpu.sync_copy(ones_v.at[c], shared.at[idx_v.at[c]], add=True)
        plsc.subcore_barrier()

        pltpu.sync_copy(
            shared.at[pl.ds(sid * SLICE, SLICE)],
            out_hbm.at[cid, pl.ds(sid * SLICE, SLICE)],
        )

    return k(idx, zeros)


def _tc_matvec(tableT, counts):
    def mv_kernel(t_ref, c_ref, o_ref):
        k = pl.program_id(0)

        @pl.when(k == 0)
        def _():
            o_ref[...] = jnp.zeros_like(o_ref)

        o_ref[...] += jax.lax.dot_general(
            t_ref[...], c_ref[...], (((1,), (1,)), ((), ())),
            preferred_element_type=jnp.float32,
            precision=jax.lax.Precision.HIGHEST,
        )

    return pl.pallas_call(
        mv_kernel,
        grid=(NBLK,),
        in_specs=[
            pl.BlockSpec((EMBED, BK), lambda k: (0, k)),
            pl.BlockSpec((NC, BK), lambda k: (0, k)),
        ],
        out_specs=pl.BlockSpec((EMBED, NC), lambda k: (0, 0)),
        out_shape=jax.ShapeDtypeStruct((EMBED, NC), jnp.float32),
    )(tableT, counts)


def _tc_mlp(partials, W1T, b1row, W2T, b2row):
    def mlp_kernel(p_ref, w1t_ref, b1_ref, w2t_ref, b2_ref, o_ref):
        avg = jnp.sum(p_ref[...], axis=0, keepdims=True) * (1.0 / N_TOKENS)
        h = jnp.maximum(jnp.dot(avg, w1t_ref[...]) + b1_ref[...], 0.0)
        o_ref[...] = jnp.dot(h, w2t_ref[...]) + b2_ref[...]

    return pl.pallas_call(
        mlp_kernel,
        out_shape=jax.ShapeDtypeStruct((1, OUT), jnp.float32),
    )(partials, W1T, b1row, W2T, b2row)


def kernel(indices, table, W1, b1, W2, b2):
    idx = indices.astype(jnp.int32).reshape(NW, NCH, CHUNK)
    zeros = jnp.zeros((NC, BINS), jnp.float32)
    counts = _sc_hist(idx, zeros)
    tableT = table.T
    mv = _tc_matvec(tableT, counts)
    tail = jnp.sum(
        tableT[:, TAIL:][:, None, :] * counts[None, :, TAIL:VOCAB], axis=2
    )
    partials = (mv + tail).T
    logits = _tc_mlp(partials, W1.T, b1[None, :], W2.T, b2[None, :])
    return logits.reshape(OUT)


# speedup vs baseline: 1.5919x; 1.5919x over previous
import functools

import jax
import jax.numpy as jnp
from jax import lax
from jax.experimental import pallas as pl
from jax.experimental.pallas import tpu as pltpu
from jax.experimental.pallas import tpu_sc as plsc

EMBED = 64
HIDDEN = 256
OUT = 2
N_TOKENS = 16384
VOCAB = 1000000

NC = 2
NS = 16
NW = NC * NS
PER_TILE = N_TOKENS // NW
CHUNK = 128
NCH = PER_TILE // CHUNK
LANES = 16
BINS = 1 << 20
SLICE = BINS // NS
BK = 23808
NBLK = 42
TAIL = NBLK * BK


def _sc_hist(idx, zeros):
    mesh = plsc.VectorSubcoreMesh(core_axis_name="c", subcore_axis_name="s")

    @functools.partial(
        pl.kernel,
        mesh=mesh,
        out_type=jax.ShapeDtypeStruct((NC, BINS), jnp.float32),
        scratch_types=[
            pltpu.VMEM((NCH, CHUNK), jnp.int32),
            pltpu.VMEM((NCH, CHUNK), jnp.float32),
            pltpu.VMEM_SHARED((BINS,), jnp.float32),
            pltpu.SemaphoreType.DMA,
        ],
    )
    def k(idx_hbm, zeros_hbm, out_hbm, idx_v, ones_v, shared, sem):
        cid = lax.axis_index("c")
        sid = lax.axis_index("s")
        wid = sid * NC + cid

        pltpu.sync_copy(idx_hbm.at[wid], idx_v)
        for c in range(NCH):
            for g in range(CHUNK // LANES):
                ones_v[c, pl.ds(g * LANES, LANES)] = jnp.ones((LANES,), jnp.float32)

        pltpu.sync_copy(
            zeros_hbm.at[cid, pl.ds(sid * SLICE, SLICE)],
            shared.at[pl.ds(sid * SLICE, SLICE)],
        )
        plsc.subcore_barrier()

        for c in range(NCH):
            pltpu.sync_copy(ones_v.at[c], shared.at[idx_v.at[c]], add=True)
        plsc.subcore_barrier()

        pltpu.sync_copy(
            shared.at[pl.ds(sid * SLICE, SLICE)],
            out_hbm.at[cid, pl.ds(sid * SLICE, SLICE)],
        )

    return k(idx, zeros)


def _tc_matvec(tableT, counts):
    def mv_kernel(t_ref, c_ref, o_ref):
        k = pl.program_id(0)

        @pl.when(k == 0)
        def _():
            o_ref[...] = jnp.zeros_like(o_ref)

        o_ref[...] += jnp.sum(t_ref[...] * c_ref[...], axis=1, keepdims=True)

    return pl.pallas_call(
        mv_kernel,
        grid=(NBLK,),
        in_specs=[
            pl.BlockSpec((EMBED, BK), lambda k: (0, k)),
            pl.BlockSpec((1, BK), lambda k: (0, k)),
        ],
        out_specs=pl.BlockSpec((EMBED, 1), lambda k: (0, 0)),
        out_shape=jax.ShapeDtypeStruct((EMBED, 1), jnp.float32),
    )(tableT, counts)


def _tc_mlp(partials, W1T, b1row, W2T, b2row):
    def mlp_kernel(p_ref, w1t_ref, b1_ref, w2t_ref, b2_ref, o_ref):
        avg = jnp.sum(p_ref[...], axis=0, keepdims=True) * (1.0 / N_TOKENS)
        h = jnp.maximum(jnp.dot(avg, w1t_ref[...]) + b1_ref[...], 0.0)
        o_ref[...] = jnp.dot(h, w2t_ref[...]) + b2_ref[...]

    return pl.pallas_call(
        mlp_kernel,
        out_shape=jax.ShapeDtypeStruct((1, OUT), jnp.float32),
    )(partials, W1T, b1row, W2T, b2row)


def kernel(indices, table, W1, b1, W2, b2):
    idx = indices.astype(jnp.int32).reshape(NW, NCH, CHUNK)
    zeros = jnp.zeros((NC, BINS), jnp.float32)
    counts2 = _sc_hist(idx, zeros)
    counts = jnp.sum(counts2, axis=0, keepdims=True)
    tableT = table.T
    mv = _tc_matvec(tableT, counts)
    tail = jnp.sum(
        tableT[:, TAIL:] * counts[:, TAIL:VOCAB], axis=1, keepdims=True
    )
    partials = (mv + tail).T
    logits = _tc_mlp(partials, W1.T, b1[None, :], W2.T, b2[None, :])
    return logits.reshape(OUT)


# speedup vs baseline: 2.0753x; 1.3037x over previous
import functools

import jax
import jax.numpy as jnp
from jax import lax
from jax.experimental import pallas as pl
from jax.experimental.pallas import tpu as pltpu
from jax.experimental.pallas import tpu_sc as plsc

EMBED = 64
HIDDEN = 256
OUT = 2
N_TOKENS = 16384
VOCAB = 1000000

NC = 2
NS = 16
NW = NC * NS
PER_TILE = N_TOKENS // NW
CHUNK = 128
NCH = PER_TILE // CHUNK
LANES = 16
BINS = 1 << 20
SLICE = BINS // NS
BK = 23808
NBLK = 42
TAIL = NBLK * BK


def _sc_hist(idx, zeros):
    mesh = plsc.VectorSubcoreMesh(core_axis_name="c", subcore_axis_name="s")

    @functools.partial(
        pl.kernel,
        mesh=mesh,
        out_type=jax.ShapeDtypeStruct((NC, BINS), jnp.float32),
        scratch_types=[
            pltpu.VMEM((NCH, CHUNK), jnp.int32),
            pltpu.VMEM((NCH, CHUNK), jnp.float32),
            pltpu.VMEM_SHARED((BINS,), jnp.float32),
            pltpu.SemaphoreType.DMA,
        ],
    )
    def k(idx_hbm, zeros_hbm, out_hbm, idx_v, ones_v, shared, sem):
        cid = lax.axis_index("c")
        sid = lax.axis_index("s")
        wid = sid * NC + cid

        pltpu.sync_copy(idx_hbm.at[wid], idx_v)
        for c in range(NCH):
            for g in range(CHUNK // LANES):
                ones_v[c, pl.ds(g * LANES, LANES)] = jnp.ones((LANES,), jnp.float32)

        pltpu.sync_copy(
            zeros_hbm.at[cid, pl.ds(sid * SLICE, SLICE)],
            shared.at[pl.ds(sid * SLICE, SLICE)],
        )
        plsc.subcore_barrier()

        for c in range(NCH):
            pltpu.sync_copy(ones_v.at[c], shared.at[idx_v.at[c]], add=True)
        plsc.subcore_barrier()

        pltpu.sync_copy(
            shared.at[pl.ds(sid * SLICE, SLICE)],
            out_hbm.at[cid, pl.ds(sid * SLICE, SLICE)],
        )

    return k(idx, zeros)


def _tc_matvec(tableT, counts):
    def mv_kernel(t_ref, c_ref, o_ref):
        k = pl.program_id(0)

        @pl.when(k == 0)
        def _():
            o_ref[...] = jnp.zeros_like(o_ref)

        c = c_ref[pl.ds(0, 1), :] + c_ref[pl.ds(1, 1), :]
        o_ref[...] += jnp.sum(t_ref[...] * c, axis=1, keepdims=True)

    return pl.pallas_call(
        mv_kernel,
        grid=(NBLK,),
        in_specs=[
            pl.BlockSpec((EMBED, BK), lambda k: (0, k)),
            pl.BlockSpec((NC, BK), lambda k: (0, k)),
        ],
        out_specs=pl.BlockSpec((EMBED, 1), lambda k: (0, 0)),
        out_shape=jax.ShapeDtypeStruct((EMBED, 1), jnp.float32),
    )(tableT, counts)


def _tc_mlp(partials, W1T, b1row, W2T, b2row):
    def mlp_kernel(p_ref, w1t_ref, b1_ref, w2t_ref, b2_ref, o_ref):
        avg = jnp.sum(p_ref[...], axis=0, keepdims=True) * (1.0 / N_TOKENS)
        h = jnp.maximum(jnp.dot(avg, w1t_ref[...]) + b1_ref[...], 0.0)
        o_ref[...] = jnp.dot(h, w2t_ref[...]) + b2_ref[...]

    return pl.pallas_call(
        mlp_kernel,
        out_shape=jax.ShapeDtypeStruct((1, OUT), jnp.float32),
    )(partials, W1T, b1row, W2T, b2row)


def kernel(indices, table, W1, b1, W2, b2):
    idx = indices.astype(jnp.int32).reshape(NW, NCH, CHUNK)
    zeros = jnp.zeros((NC, BINS), jnp.float32)
    counts = _sc_hist(idx, zeros)
    tableT = table.T
    mv = _tc_matvec(tableT, counts)
    ctail = counts[0, TAIL:VOCAB] + counts[1, TAIL:VOCAB]
    tail = jnp.sum(
        tableT[:, TAIL:] * ctail[None, :], axis=1, keepdims=True
    )
    partials = (mv + tail).T
    logits = _tc_mlp(partials, W1.T, b1[None, :], W2.T, b2[None, :])
    return logits.reshape(OUT)
